# baseline (device time: 189874 ns/iter reference)
import functools
import os

import jax
import jax.numpy as jnp
from jax import lax
from jax.experimental import pallas as pl
from jax.experimental.pallas import tpu as pltpu

N_DEV = 4

SIZES = [int(s) for s in "2048,1024,512".split(",")]


def kernel(A, B):
    m, k = A.shape
    _, n = B.shape

    f32 = jnp.float32
    bf16 = jnp.bfloat16
    nmsg = len(SIZES)
    maxr = max(SIZES)

    def body(a_ref, b_ref, out_ref, sb, rb, ssems, rsems):
        my = lax.axis_index("i")
        px = 3 - my
        left = (my + N_DEV - 1) % N_DEV
        right = (my + 1) % N_DEV

        barrier_sem = pltpu.get_barrier_semaphore()
        for nbr in (left, right):
            pl.semaphore_signal(
                barrier_sem, inc=1,
                device_id=(nbr,), device_id_type=pl.DeviceIdType.MESH,
            )
        pl.semaphore_wait(barrier_sem, 2)

        for i, rows in enumerate(SIZES):
            rdma = pltpu.make_async_remote_copy(
                src_ref=sb.at[pl.ds(0, rows), :],
                dst_ref=rb.at[pl.ds(0, rows), :],
                send_sem=ssems.at[i],
                recv_sem=rsems.at[i],
                device_id=(px,),
                device_id_type=pl.DeviceIdType.MESH,
            )
            rdma.start()
            rdma.wait()

        out_ref[0:8, 0:128] = rb[0:8, 0:128].astype(f32)

        @functools.partial(
            pl.run_scoped, second_barrier=pltpu.SemaphoreType.REGULAR
        )
        def _(second_barrier):
            for nbr in (left, right):
                pl.semaphore_signal(
                    second_barrier, inc=1,
                    device_id=(nbr,), device_id_type=pl.DeviceIdType.MESH,
                )
            pl.semaphore_wait(second_barrier, 2)

    return pl.pallas_call(
        body,
        out_shape=jax.ShapeDtypeStruct((m, n), f32),
        in_specs=[
            pl.BlockSpec(memory_space=pltpu.VMEM),
            pl.BlockSpec(memory_space=pltpu.VMEM),
        ],
        out_specs=pl.BlockSpec(memory_space=pltpu.VMEM),
        scratch_shapes=[
            pltpu.VMEM((maxr, n), bf16),
            pltpu.VMEM((maxr, n), bf16),
            pltpu.SemaphoreType.DMA((nmsg,)),
            pltpu.SemaphoreType.DMA((nmsg,)),
        ],
        compiler_params=pltpu.CompilerParams(
            collective_id=0, vmem_limit_bytes=100 * 1024 * 1024
        ),
    )(A, B)


# device time: 104721 ns/iter; 1.8131x vs baseline; 1.8131x over previous
import functools

import jax
import jax.numpy as jnp
from jax import lax
from jax.experimental import pallas as pl
from jax.experimental.pallas import tpu as pltpu

N_DEV = 4


def kernel(A, B):
    m, k = A.shape
    _, n = B.shape
    hm = m // 2
    qm = m // 4
    half = n // 2

    f32 = jnp.float32
    bf16 = jnp.bfloat16

    def body(a_ref, b_ref, out_ref, sb1, rb1, sb2, rb2, sb3, rb3, rb4,
             s1s, s1r, s2s, s2r, s3s, s3r, s4s, s4r):
        my = lax.axis_index("i")
        x = my // 2
        y = (my % 2) ^ x
        px = 3 - my
        py = my ^ 1
        left = (my + N_DEV - 1) % N_DEV
        right = (my + 1) % N_DEV

        barrier_sem = pltpu.get_barrier_semaphore()
        for nbr in (left, right):
            pl.semaphore_signal(
                barrier_sem, inc=1,
                device_id=(nbr,), device_id_type=pl.DeviceIdType.MESH,
            )

        cols = (slice(0, half), slice(half, n))
        maj = (x, y)
        mnr = (y, x)
        partner1 = (px, py)
        partner2 = (py, px)
        own_h = tuple(c * hm for c in maj)
        oth_h = tuple((1 - c) * hm for c in maj)
        own_q = tuple(own_h[h] + mnr[h] * qm for h in range(2))
        oth_q = tuple(own_h[h] + (1 - mnr[h]) * qm for h in range(2))
        r4a = tuple(oth_h[h] + mnr[h] * qm for h in range(2))
        r4b = tuple(oth_h[h] + (1 - mnr[h]) * qm for h in range(2))

        def mesh_copy(src, dst, ssem, rsem, dev):
            return pltpu.make_async_remote_copy(
                src_ref=src, dst_ref=dst, send_sem=ssem, recv_sem=rsem,
                device_id=(dev,), device_id_type=pl.DeviceIdType.MESH,
            )

        st1 = [[mesh_copy(sb1.at[h, pl.ds(u * qm, qm), :],
                          rb1.at[h, pl.ds(u * qm, qm), :],
                          s1s.at[h, u], s1r.at[h, u], partner1[h])
                for u in range(2)] for h in range(2)]
        st2 = [mesh_copy(sb2.at[h], rb2.at[h], s2s.at[h], s2r.at[h],
                         partner2[h]) for h in range(2)]
        st3 = [mesh_copy(sb3.at[h], rb3.at[h], s3s.at[h], s3r.at[h],
                         partner2[h]) for h in range(2)]
        st4 = [[mesh_copy((sb3 if u == 0 else rb3).at[h],
                          rb4.at[h, pl.ds(u * qm, qm), :],
                          s4s.at[h, u], s4r.at[h, u], partner1[h])
                for u in range(2)] for h in range(2)]

        def half_dot(row_start, h):
            return jnp.dot(
                a_ref[pl.ds(row_start, qm), :], b_ref[:, cols[h]],
                preferred_element_type=f32,
            )

        for u in range(2):
            for h in range(2):
                sb1[h, pl.ds(u * qm, qm), :] = (
                    half_dot(oth_h[h] + u * qm, h).astype(bf16)
                )

        pl.semaphore_wait(barrier_sem, 2)
        for u in range(2):
            for h in range(2):
                st1[h][u].start()

        for h in range(2):
            for u in range(2):
                out_ref[pl.ds(own_h[h] + u * qm, qm), cols[h]] = (
                    half_dot(own_h[h] + u * qm, h)
                )

        for h in range(2):
            for u in range(2):
                st1[h][u].wait_recv()
                rows = pl.ds(own_h[h] + u * qm, qm)
                out_ref[rows, cols[h]] = (
                    out_ref[rows, cols[h]]
                    + rb1[h, pl.ds(u * qm, qm), :].astype(f32)
                )
            sb2[h] = out_ref[pl.ds(oth_q[h], qm), cols[h]].astype(bf16)
            st2[h].start()

        for h in range(2):
            st2[h].wait_recv()
            rows = pl.ds(own_q[h], qm)
            acc = out_ref[rows, cols[h]] + rb2[h].astype(f32)
            out_ref[rows, cols[h]] = acc
            sb3[h] = acc.astype(bf16)
            st3[h].start()
            st4[h][0].start()

        for h in range(2):
            st3[h].wait_recv()
            st4[h][1].start()
            out_ref[pl.ds(oth_q[h], qm), cols[h]] = rb3[h].astype(f32)

        for h in range(2):
            for u in range(2):
                st4[h][u].wait_recv()
            out_ref[pl.ds(r4a[h], qm), cols[h]] = (
                rb4[h, pl.ds(0, qm), :].astype(f32)
            )
            out_ref[pl.ds(r4b[h], qm), cols[h]] = (
                rb4[h, pl.ds(qm, qm), :].astype(f32)
            )

        for h in range(2):
            for u in range(2):
                st1[h][u].wait_send()
                st4[h][u].wait_send()
            st2[h].wait_send()
            st3[h].wait_send()

    return pl.pallas_call(
        body,
        out_shape=jax.ShapeDtypeStruct((m, n), f32),
        in_specs=[
            pl.BlockSpec(memory_space=pltpu.VMEM),
            pl.BlockSpec(memory_space=pltpu.VMEM),
        ],
        out_specs=pl.BlockSpec(memory_space=pltpu.VMEM),
        scratch_shapes=[
            pltpu.VMEM((2, hm, half), bf16),
            pltpu.VMEM((2, hm, half), bf16),
            pltpu.VMEM((2, qm, half), bf16),
            pltpu.VMEM((2, qm, half), bf16),
            pltpu.VMEM((2, qm, half), bf16),
            pltpu.VMEM((2, qm, half), bf16),
            pltpu.VMEM((2, hm, half), bf16),
            pltpu.SemaphoreType.DMA((2, 2)),
            pltpu.SemaphoreType.DMA((2, 2)),
            pltpu.SemaphoreType.DMA((2,)),
            pltpu.SemaphoreType.DMA((2,)),
            pltpu.SemaphoreType.DMA((2,)),
            pltpu.SemaphoreType.DMA((2,)),
            pltpu.SemaphoreType.DMA((2, 2)),
            pltpu.SemaphoreType.DMA((2, 2)),
        ],
        compiler_params=pltpu.CompilerParams(
            collective_id=0, vmem_limit_bytes=100 * 1024 * 1024
        ),
    )(A, B)


# device time: 104574 ns/iter; 1.8157x vs baseline; 1.0014x over previous
import functools

import jax
import jax.numpy as jnp
from jax import lax
from jax.experimental import pallas as pl
from jax.experimental.pallas import tpu as pltpu

N_DEV = 4


def kernel(A, B):
    m, k = A.shape
    _, n = B.shape
    hm = m // 2
    qm = m // 4
    half = n // 2

    f32 = jnp.float32
    bf16 = jnp.bfloat16

    def body(a_ref, b_ref, out_ref, sb1, rb1, sb2, rb2, sb3, rb3, rb4,
             s1s, s1r, s2s, s2r, s3s, s3r, s4s, s4r):
        my = lax.axis_index("i")
        x = my // 2
        y = (my % 2) ^ x
        px = 3 - my
        py = my ^ 1
        left = (my + N_DEV - 1) % N_DEV
        right = (my + 1) % N_DEV

        barrier_sem = pltpu.get_barrier_semaphore()
        for nbr in (left, right):
            pl.semaphore_signal(
                barrier_sem, inc=1,
                device_id=(nbr,), device_id_type=pl.DeviceIdType.MESH,
            )

        cols = (slice(0, half), slice(half, n))
        maj = (x, y)
        mnr = (y, x)
        partner1 = (px, py)
        partner2 = (py, px)
        own_h = tuple(c * hm for c in maj)
        oth_h = tuple((1 - c) * hm for c in maj)
        own_q = tuple(own_h[h] + mnr[h] * qm for h in range(2))
        oth_q = tuple(own_h[h] + (1 - mnr[h]) * qm for h in range(2))
        r4a = tuple(oth_h[h] + mnr[h] * qm for h in range(2))
        r4b = tuple(oth_h[h] + (1 - mnr[h]) * qm for h in range(2))

        def mesh_copy(src, dst, ssem, rsem, dev):
            return pltpu.make_async_remote_copy(
                src_ref=src, dst_ref=dst, send_sem=ssem, recv_sem=rsem,
                device_id=(dev,), device_id_type=pl.DeviceIdType.MESH,
            )

        st1 = [mesh_copy(sb1.at[h], rb1.at[h],
                         s1s.at[h, 0], s1r.at[h, 0], partner1[h])
               for h in range(2)]
        st2 = [mesh_copy(sb2.at[h], rb2.at[h], s2s.at[h], s2r.at[h],
                         partner2[h]) for h in range(2)]
        st3 = [mesh_copy(sb3.at[h], rb3.at[h], s3s.at[h], s3r.at[h],
                         partner2[h]) for h in range(2)]
        st4 = [[mesh_copy((sb3 if u == 0 else rb3).at[h],
                          rb4.at[h, pl.ds(u * qm, qm), :],
                          s4s.at[h, u], s4r.at[h, u], partner1[h])
                for u in range(2)] for h in range(2)]

        def half_dot(row_start, h):
            return jnp.dot(
                a_ref[pl.ds(row_start, qm), :], b_ref[:, cols[h]],
                preferred_element_type=f32,
            )

        for u in range(2):
            for h in range(2):
                sb1[h, pl.ds(u * qm, qm), :] = (
                    half_dot(oth_h[h] + u * qm, h).astype(bf16)
                )

        pl.semaphore_wait(barrier_sem, 2)
        for h in range(2):
            st1[h].start()

        for h in range(2):
            for u in range(2):
                out_ref[pl.ds(own_h[h] + u * qm, qm), cols[h]] = (
                    half_dot(own_h[h] + u * qm, h)
                )

        for h in range(2):
            st1[h].wait_recv()
            rows = pl.ds(oth_q[h], qm)
            rbrows = pl.ds(oth_q[h] - own_h[h], qm)
            acc = out_ref[rows, cols[h]] + rb1[h, rbrows, :].astype(f32)
            sb2[h] = acc.astype(bf16)
            st2[h].start()
            out_ref[rows, cols[h]] = acc
        for h in range(2):
            rows = pl.ds(own_q[h], qm)
            rbrows = pl.ds(own_q[h] - own_h[h], qm)
            out_ref[rows, cols[h]] = (
                out_ref[rows, cols[h]] + rb1[h, rbrows, :].astype(f32)
            )

        for h in range(2):
            st2[h].wait_recv()
            rows = pl.ds(own_q[h], qm)
            acc = out_ref[rows, cols[h]] + rb2[h].astype(f32)
            out_ref[rows, cols[h]] = acc
            sb3[h] = acc.astype(bf16)
            st3[h].start()
            st4[h][0].start()

        for h in range(2):
            st3[h].wait_recv()
            st4[h][1].start()
            out_ref[pl.ds(oth_q[h], qm), cols[h]] = rb3[h].astype(f32)

        for h in range(2):
            for u in range(2):
                st4[h][u].wait_recv()
            out_ref[pl.ds(r4a[h], qm), cols[h]] = (
                rb4[h, pl.ds(0, qm), :].astype(f32)
            )
            out_ref[pl.ds(r4b[h], qm), cols[h]] = (
                rb4[h, pl.ds(qm, qm), :].astype(f32)
            )

        for h in range(2):
            st1[h].wait_send()
            for u in range(2):
                st4[h][u].wait_send()
            st2[h].wait_send()
            st3[h].wait_send()

    return pl.pallas_call(
        body,
        out_shape=jax.ShapeDtypeStruct((m, n), f32),
        in_specs=[
            pl.BlockSpec(memory_space=pltpu.VMEM),
            pl.BlockSpec(memory_space=pltpu.VMEM),
        ],
        out_specs=pl.BlockSpec(memory_space=pltpu.VMEM),
        scratch_shapes=[
            pltpu.VMEM((2, hm, half), bf16),
            pltpu.VMEM((2, hm, half), bf16),
            pltpu.VMEM((2, qm, half), bf16),
            pltpu.VMEM((2, qm, half), bf16),
            pltpu.VMEM((2, qm, half), bf16),
            pltpu.VMEM((2, qm, half), bf16),
            pltpu.VMEM((2, hm, half), bf16),
            pltpu.SemaphoreType.DMA((2, 2)),
            pltpu.SemaphoreType.DMA((2, 2)),
            pltpu.SemaphoreType.DMA((2,)),
            pltpu.SemaphoreType.DMA((2,)),
            pltpu.SemaphoreType.DMA((2,)),
            pltpu.SemaphoreType.DMA((2,)),
            pltpu.SemaphoreType.DMA((2, 2)),
            pltpu.SemaphoreType.DMA((2, 2)),
        ],
        compiler_params=pltpu.CompilerParams(
            collective_id=0, vmem_limit_bytes=100 * 1024 * 1024
        ),
    )(A, B)


# device time: 94852 ns/iter; 2.0018x vs baseline; 1.1025x over previous
import jax
import jax.numpy as jnp
from jax import lax
from jax.experimental import pallas as pl
from jax.experimental.pallas import tpu as pltpu

N_DEV = 4
N_SUB = 2


def kernel(A, B):
    m, k = A.shape
    _, n = B.shape
    mc = m // N_DEV
    sub = mc // N_SUB
    half = n // 2

    f32 = jnp.float32
    bf16 = jnp.bfloat16

    def body(a_ref, b_ref, out_hbm, res, rs0_buf, rs_buf, ag_buf,
             rs_send_sems, rs_recv_sems, ag_send_sems, ag_recv_sems,
             cp_sems):
        my = lax.axis_index("i")
        left = (my + N_DEV - 1) % N_DEV
        right = (my + 1) % N_DEV
        ring_dst = (right, left)

        barrier_sem = pltpu.get_barrier_semaphore()
        for nbr in (left, right):
            pl.semaphore_signal(
                barrier_sem, inc=1,
                device_id=(nbr,), device_id_type=pl.DeviceIdType.MESH,
            )

        def srows(c, u):
            return pl.ds(c * mc + u * sub, sub)

        def crows(c):
            return pl.ds(c * mc, mc)

        cols = (slice(0, half), slice(half, n))

        def half_dot(c, u, r):
            return jnp.dot(
                a_ref[srows(c, u), :], b_ref[:, cols[r]],
                preferred_element_type=f32,
            )

        def make_rs(s, u, r):
            src = rs0_buf.at[r] if s == 0 else rs_buf.at[r, s - 1]
            return pltpu.make_async_remote_copy(
                src_ref=src.at[pl.ds(u * sub, sub), :],
                dst_ref=rs_buf.at[r, s, pl.ds(u * sub, sub), :],
                send_sem=rs_send_sems.at[r, s, u],
                recv_sem=rs_recv_sems.at[r, s, u],
                device_id=(ring_dst[r],),
                device_id_type=pl.DeviceIdType.MESH,
            )

        def make_ag(t, u, r):
            return pltpu.make_async_remote_copy(
                src_ref=ag_buf.at[r, t, pl.ds(u * sub, sub), :],
                dst_ref=ag_buf.at[r, t + 1, pl.ds(u * sub, sub), :],
                send_sem=ag_send_sems.at[r, t, u],
                recv_sem=ag_recv_sems.at[r, t, u],
                device_id=(ring_dst[r],),
                device_id_type=pl.DeviceIdType.MESH,
            )

        rs = [[[make_rs(s, u, r) for r in range(2)] for u in range(N_SUB)]
              for s in range(N_DEV - 1)]
        ag = [[[make_ag(t, u, r) for r in range(2)] for u in range(N_SUB)]
              for t in range(N_DEV - 1)]

        def flush(c, r, sem_idx):
            cp = pltpu.make_async_copy(
                res.at[crows(c), cols[r]],
                out_hbm.at[crows(c), cols[r]],
                cp_sems.at[sem_idx],
            )
            cp.start()
            return cp

        for u in range(N_SUB):
            for r in range(2):
                rs0_buf[r, pl.ds(u * sub, sub), :] = (
                    half_dot(my, u, r).astype(bf16)
                )
        pl.semaphore_wait(barrier_sem, 2)
        for u in range(N_SUB):
            for r in range(2):
                rs[0][u][r].start()

        rs_c = [((my + N_DEV - 1 - s) % N_DEV, (my + 1 + s) % N_DEV)
                for s in range(N_DEV - 1)]

        copies = []

        def acc(s, u):
            for r in range(2):
                rs_buf[r, s, pl.ds(u * sub, sub), :] = (
                    rs_buf[r, s, pl.ds(u * sub, sub), :].astype(f32)
                    + res[srows(rs_c[s][r], u), cols[r]]
                ).astype(bf16)
                rs[s + 1][u][r].start()

        def final_acc(u):
            s = N_DEV - 2
            for r in range(2):
                a = (
                    rs_buf[r, s, pl.ds(u * sub, sub), :].astype(f32)
                    + res[srows(rs_c[s][r], u), cols[r]]
                )
                res[srows(rs_c[s][r], u), cols[r]] = a
                ag_buf[r, 0, pl.ds(u * sub, sub), :] = a.astype(bf16)
                ag[0][u][r].start()
            if u == N_SUB - 1:
                for r in range(2):
                    copies.append(flush(rs_c[s][r], r, r))

        windows = [(0, 0), (0, 1), (1, 0), (1, 1), (2, 0), (2, 1)]
        for s, u in windows:
            for r in range(2):
                res[srows(rs_c[s][r], u), cols[r]] = half_dot(
                    rs_c[s][r], u, r
                )
            for r in range(2):
                rs[s][u][r].wait_recv()
            if s < N_DEV - 2:
                acc(s, u)
            else:
                final_acc(u)

        for t in range(N_DEV - 1):
            c = ((my + N_DEV - t) % N_DEV, (my + t) % N_DEV)
            for u in range(N_SUB):
                for r in range(2):
                    ag[t][u][r].wait_recv()
                if t < N_DEV - 2:
                    for r in range(2):
                        ag[t + 1][u][r].start()
                for r in range(2):
                    res[srows(c[r], u), cols[r]] = (
                        ag_buf[r, t + 1, pl.ds(u * sub, sub), :].astype(f32)
                    )
            for r in range(2):
                copies.append(flush(c[r], r, 2 + 2 * t + r))

        for cp in copies:
            cp.wait()
        for group in (rs, ag):
            for hop in group:
                for u_list in hop:
                    for rdma in u_list:
                        rdma.wait_send()

    return pl.pallas_call(
        body,
        out_shape=jax.ShapeDtypeStruct((m, n), f32),
        in_specs=[
            pl.BlockSpec(memory_space=pltpu.VMEM),
            pl.BlockSpec(memory_space=pltpu.VMEM),
        ],
        out_specs=pl.BlockSpec(memory_space=pl.ANY),
        scratch_shapes=[
            pltpu.VMEM((m, n), f32),
            pltpu.VMEM((2, mc, half), bf16),
            pltpu.VMEM((2, N_DEV - 1, mc, half), bf16),
            pltpu.VMEM((2, N_DEV, mc, half), bf16),
            pltpu.SemaphoreType.DMA((2, N_DEV - 1, N_SUB)),
            pltpu.SemaphoreType.DMA((2, N_DEV - 1, N_SUB)),
            pltpu.SemaphoreType.DMA((2, N_DEV - 1, N_SUB)),
            pltpu.SemaphoreType.DMA((2, N_DEV - 1, N_SUB)),
            pltpu.SemaphoreType.DMA((8,)),
        ],
        compiler_params=pltpu.CompilerParams(
            collective_id=0, vmem_limit_bytes=100 * 1024 * 1024
        ),
    )(A, B)


# device time: 94839 ns/iter; 2.0021x vs baseline; 1.0001x over previous
import jax
import jax.numpy as jnp
from jax import lax
from jax.experimental import pallas as pl
from jax.experimental.pallas import tpu as pltpu

N_DEV = 4
N_SUB = 2


def kernel(A, B):
    m, k = A.shape
    _, n = B.shape
    mc = m // N_DEV
    sub = mc // N_SUB
    half = n // 2

    f32 = jnp.float32
    bf16 = jnp.bfloat16

    def body(a_ref, b_ref, out_hbm, res, rs0_buf, rs_buf, ag_buf,
             rs_send_sems, rs_recv_sems, ag_send_sems, ag_recv_sems,
             cp_sems):
        my = lax.axis_index("i")
        left = (my + N_DEV - 1) % N_DEV
        right = (my + 1) % N_DEV
        ring_dst = (right, left)

        barrier_sem = pltpu.get_barrier_semaphore()
        for nbr in (left, right):
            pl.semaphore_signal(
                barrier_sem, inc=1,
                device_id=(nbr,), device_id_type=pl.DeviceIdType.MESH,
            )

        def srows(c, u):
            return pl.ds(c * mc + u * sub, sub)

        def crows(c):
            return pl.ds(c * mc, mc)

        cols = (slice(0, half), slice(half, n))

        def half_dot(c, u, r):
            return jnp.dot(
                a_ref[srows(c, u), :], b_ref[:, cols[r]],
                preferred_element_type=f32,
            )

        def make_rs(s, u, r):
            src = rs0_buf.at[r] if s == 0 else rs_buf.at[r, s - 1]
            return pltpu.make_async_remote_copy(
                src_ref=src.at[pl.ds(u * sub, sub), :],
                dst_ref=rs_buf.at[r, s, pl.ds(u * sub, sub), :],
                send_sem=rs_send_sems.at[r, s, u],
                recv_sem=rs_recv_sems.at[r, s, u],
                device_id=(ring_dst[r],),
                device_id_type=pl.DeviceIdType.MESH,
            )

        def make_ag(t, u, r):
            return pltpu.make_async_remote_copy(
                src_ref=ag_buf.at[r, t, pl.ds(u * sub, sub), :],
                dst_ref=ag_buf.at[r, t + 1, pl.ds(u * sub, sub), :],
                send_sem=ag_send_sems.at[r, t, u],
                recv_sem=ag_recv_sems.at[r, t, u],
                device_id=(ring_dst[r],),
                device_id_type=pl.DeviceIdType.MESH,
            )

        rs = [[[make_rs(s, u, r) for r in range(2)] for u in range(N_SUB)]
              for s in range(N_DEV - 1)]
        ag = [[[make_ag(t, u, r) for r in range(2)] for u in range(N_SUB)]
              for t in range(N_DEV - 1)]

        def flush(c, r, sem_idx):
            cp = pltpu.make_async_copy(
                res.at[crows(c), cols[r]],
                out_hbm.at[crows(c), cols[r]],
                cp_sems.at[sem_idx],
            )
            cp.start()
            return cp

        pl.semaphore_wait(barrier_sem, 2)
        for u in range(N_SUB):
            for r in range(2):
                rs0_buf[r, pl.ds(u * sub, sub), :] = (
                    half_dot(my, u, r).astype(bf16)
                )
                rs[0][u][r].start()

        rs_c = [((my + N_DEV - 1 - s) % N_DEV, (my + 1 + s) % N_DEV)
                for s in range(N_DEV - 1)]

        copies = []

        def acc(s, u):
            for r in range(2):
                rs_buf[r, s, pl.ds(u * sub, sub), :] = (
                    rs_buf[r, s, pl.ds(u * sub, sub), :].astype(f32)
                    + res[srows(rs_c[s][r], u), cols[r]]
                ).astype(bf16)
                rs[s + 1][u][r].start()

        def final_acc(u):
            s = N_DEV - 2
            for r in range(2):
                a = (
                    rs_buf[r, s, pl.ds(u * sub, sub), :].astype(f32)
                    + res[srows(rs_c[s][r], u), cols[r]]
                )
                res[srows(rs_c[s][r], u), cols[r]] = a
                ag_buf[r, 0, pl.ds(u * sub, sub), :] = a.astype(bf16)
                ag[0][u][r].start()
            if u == N_SUB - 1:
                for r in range(2):
                    copies.append(flush(rs_c[s][r], r, r))

        windows = [(s, u) for s in range(N_DEV - 1) for u in range(N_SUB)]
        for s, u in windows:
            for r in range(2):
                res[srows(rs_c[s][r], u), cols[r]] = half_dot(
                    rs_c[s][r], u, r
                )
            for r in range(2):
                rs[s][u][r].wait_recv()
            if s < N_DEV - 2:
                acc(s, u)
            else:
                final_acc(u)

        for t in range(N_DEV - 1):
            c = ((my + N_DEV - t) % N_DEV, (my + t) % N_DEV)
            for u in range(N_SUB):
                for r in range(2):
                    ag[t][u][r].wait_recv()
                if t < N_DEV - 2:
                    for r in range(2):
                        ag[t + 1][u][r].start()
                for r in range(2):
                    res[srows(c[r], u), cols[r]] = (
                        ag_buf[r, t + 1, pl.ds(u * sub, sub), :].astype(f32)
                    )
            for r in range(2):
                copies.append(flush(c[r], r, 2 + 2 * t + r))

        for cp in copies:
            cp.wait()
        for group in (rs, ag):
            for hop in group:
                for u_list in hop:
                    for rdma in u_list:
                        rdma.wait_send()

    return pl.pallas_call(
        body,
        out_shape=jax.ShapeDtypeStruct((m, n), f32),
        in_specs=[
            pl.BlockSpec(memory_space=pltpu.VMEM),
            pl.BlockSpec(memory_space=pltpu.VMEM),
        ],
        out_specs=pl.BlockSpec(memory_space=pl.ANY),
        scratch_shapes=[
            pltpu.VMEM((m, n), f32),
            pltpu.VMEM((2, mc, half), bf16),
            pltpu.VMEM((2, N_DEV - 1, mc, half), bf16),
            pltpu.VMEM((2, N_DEV, mc, half), bf16),
            pltpu.SemaphoreType.DMA((2, N_DEV - 1, N_SUB)),
            pltpu.SemaphoreType.DMA((2, N_DEV - 1, N_SUB)),
            pltpu.SemaphoreType.DMA((2, N_DEV - 1, N_SUB)),
            pltpu.SemaphoreType.DMA((2, N_DEV - 1, N_SUB)),
            pltpu.SemaphoreType.DMA((8,)),
        ],
        compiler_params=pltpu.CompilerParams(
            collective_id=0, vmem_limit_bytes=100 * 1024 * 1024
        ),
    )(A, B)
